# baseline (speedup 1.0000x reference)
import functools

import jax
import jax.numpy as jnp
from jax import lax
from jax.experimental import pallas as pl
from jax.experimental.pallas import tpu as pltpu
from jax.experimental.pallas import tpu_sc as plsc

N = 50000
D = 64
E = 800000

NQ = 4
FQ = 16
NT = 16
NC = 2

CHUNK = 128
CPB = 16
BLOCKS = 25
RB = 4
EPT = CHUNK * CPB * BLOCKS
EP = EPT * NT
PAD = EP - E
IROWS = EP // CHUNK

NP = 50048
NPT = NP // NT
WCHUNK = 136
NW = NPT // WCHUNK
GOFFS = tuple(range(0, WCHUNK - 15, 16)) + (WCHUNK - 16,)


def _sc_body(hq, is_dst, is_src, out_hbm,
             acc, rst, norm_d, norm_s, ib_g, ib_s, rows, ones, zb, inb,
             outb, nbuf, sems):
  cid = lax.axis_index("c")
  sid = lax.axis_index("s")
  sem_g, sem_f, sem_o, sem_i, sem_w, sem_z = sems

  def _init_ones(i, _):
    ones[i, :] = jnp.ones((FQ,), jnp.float32)
    return 0
  lax.fori_loop(0, CHUNK, _init_ones, 0)

  def _init_z(i, _):
    zb[i, :] = jnp.zeros((FQ,), jnp.float32)
    return 0
  lax.fori_loop(0, WCHUNK, _init_z, 0)

  for z in range(NW):
    pltpu.sync_copy(zb, acc.at[pl.ds(sid * NPT + z * WCHUNK, WCHUNK)])
    pltpu.sync_copy(zb, rst.at[pl.ds(sid * NPT + z * WCHUNK, WCHUNK)])
  plsc.subcore_barrier()

  zero16 = jnp.zeros((16,), jnp.int32)
  iota16 = lax.iota(jnp.int32, 16)

  def deg_prologue():
    def block(b, _):
      srow0 = sid * (BLOCKS * CPB) + b * CPB
      ci0 = pltpu.async_copy(is_dst.at[pl.ds(srow0, CPB)], ib_s, sem_i[0])
      ci1 = pltpu.async_copy(is_src.at[pl.ds(srow0, CPB)], ib_g, sem_i[1])
      ci0.wait()
      ci1.wait()
      scd = [None] * RB
      scs = [None] * RB
      for j in range(CPB):
        if scd[j % RB] is not None:
          scd[j % RB].wait()
        scd[j % RB] = pltpu.async_copy(
            ones, acc.at[ib_s.at[j]], sem_o[j % RB], add=True)
        if scs[j % RB] is not None:
          scs[j % RB].wait()
        scs[j % RB] = pltpu.async_copy(
            ones, rst.at[ib_g.at[j]], sem_f[j % RB], add=True)
      for x in scd + scs:
        if x is not None:
          x.wait()
      return 0
    lax.fori_loop(0, BLOCKS, block, 0)
    plsc.subcore_barrier()

    def compact(src_ref, norm_x, rezero):
      def body(w, _):
        node0 = sid * NPT + w * WCHUNK
        pltpu.sync_copy(src_ref.at[pl.ds(node0, WCHUNK)], inb.at[0])
        for g in GOFFS:
          dv = plsc.load_gather(inb, [zero16, iota16 + g, zero16])
          nbuf[pl.ds(g, 16)] = 1.0 / jnp.maximum(dv, 1.0)
        pltpu.sync_copy(nbuf, norm_x.at[pl.ds(node0, WCHUNK)])
        if rezero:
          pltpu.sync_copy(zb, src_ref.at[pl.ds(node0, WCHUNK)])
        return 0
      lax.fori_loop(0, NW, body, 0)

    compact(acc, norm_d, True)
    compact(rst, norm_s, False)
    plsc.subcore_barrier()

  deg_prologue()

  def stage(g_tbl, ig, is2, qc, mul4):
    def xf(sl):
      if mul4:
        ib_g[sl] = jnp.minimum(ib_g[sl] * 4 + qc, 4 * N - 1)

    def block(b, _):
      row0 = sid * (BLOCKS * CPB) + b * CPB
      ci0 = pltpu.async_copy(ig.at[pl.ds(row0, CPB)], ib_g, sem_i[0])
      ci1 = pltpu.async_copy(is2.at[pl.ds(row0, CPB)], ib_s, sem_i[1])
      ci0.wait()
      gcp = [None] * RB
      scf = [None] * RB
      for r in range(RB - 1):
        for k in range(CHUNK // 16):
          xf((r, pl.ds(k * 16, 16)))
        gcp[r] = pltpu.async_copy(g_tbl.at[ib_g.at[r]], rows.at[r], sem_g[r])
      for r in range(RB - 1, CPB):
        for k in range(CHUNK // 16):
          xf((r, pl.ds(k * 16, 16)))
      ci1.wait()
      for j in range(CPB):
        jn = j + RB - 1
        if jn < CPB:
          bn = jn % RB
          if scf[bn] is not None:
            scf[bn].wait()
          gcp[bn] = pltpu.async_copy(
              g_tbl.at[ib_g.at[jn]], rows.at[bn], sem_g[bn])
        gcp[j % RB].wait()
        scf[j % RB] = pltpu.async_copy(
            rows.at[j % RB], acc.at[ib_s.at[j]], sem_f[j % RB], add=True)
      for x in scf:
        if x is not None:
          x.wait()
      return 0
    lax.fori_loop(0, BLOCKS, block, 0)

  def writeout(dst_ref, norm_x, qb):
    node00 = sid * NPT
    icp = [None, None]
    ocp = None
    zcp = [None, None]
    icp[0] = pltpu.async_copy(
        acc.at[pl.ds(node00, WCHUNK)], inb.at[0], sem_w[0])
    for w in range(NW):
      pb = w % 2
      nb = (w + 1) % 2
      node0 = node00 + w * WCHUNK
      if w + 1 < NW:
        icp[nb] = pltpu.async_copy(
            acc.at[pl.ds(node0 + WCHUNK, WCHUNK)], inb.at[nb], sem_w[nb])
      pltpu.sync_copy(norm_x.at[pl.ds(node0, WCHUNK)], nbuf)
      icp[pb].wait()
      if ocp is not None:
        ocp.wait()
      def xform(i4, _):
        for u in range(4):
          i = 4 * i4 + u
          nv = plsc.load_gather(nbuf, [zero16 + i])
          outb[i, :] = inb[pb, i, :] * nv
        return 0
      lax.fori_loop(0, WCHUNK // 4, xform, 0)
      ocp = pltpu.async_copy(
          outb, dst_ref.at[pl.ds(qb + node0, WCHUNK)], sem_w[2])
      if zcp[pb] is not None:
        zcp[pb].wait()
      zcp[pb] = pltpu.async_copy(
          zb, acc.at[pl.ds(node0, WCHUNK)], sem_z[pb])
    ocp.wait()
    for x in zcp:
      if x is not None:
        x.wait()

  def one_pass(p, _):
    q = 2 * p + cid
    qb = q * NP
    stage(hq, is_src, is_dst, q, True)
    plsc.subcore_barrier()
    writeout(rst, norm_d, 0)
    plsc.subcore_barrier()
    stage(rst, is_dst, is_src, 0, False)
    plsc.subcore_barrier()
    writeout(out_hbm, norm_s, qb)
    plsc.subcore_barrier()
    return 0

  lax.fori_loop(0, NC, one_pass, 0)


@functools.partial(jax.jit, static_argnames=())
def _run(hq, s_dst, s_src):
  mesh = plsc.VectorSubcoreMesh(core_axis_name="c", subcore_axis_name="s")
  f = pl.kernel(
      _sc_body,
      out_type=jax.ShapeDtypeStruct((NQ * NP, FQ), jnp.float32),
      mesh=mesh,
      scratch_types=[
          pltpu.VMEM_SHARED((NP, FQ), jnp.float32),
          pltpu.VMEM_SHARED((NP, FQ), jnp.float32),
          pltpu.VMEM_SHARED((NP,), jnp.float32),
          pltpu.VMEM_SHARED((NP,), jnp.float32),
          pltpu.VMEM((CPB, CHUNK), jnp.int32),
          pltpu.VMEM((CPB, CHUNK), jnp.int32),
          pltpu.VMEM((RB, CHUNK, FQ), jnp.float32),
          pltpu.VMEM((CHUNK, FQ), jnp.float32),
          pltpu.VMEM((WCHUNK, FQ), jnp.float32),
          pltpu.VMEM((2, WCHUNK, FQ), jnp.float32),
          pltpu.VMEM((WCHUNK, FQ), jnp.float32),
          pltpu.VMEM((WCHUNK,), jnp.float32),
          (
              [pltpu.SemaphoreType.DMA] * RB,
              [pltpu.SemaphoreType.DMA] * RB,
              [pltpu.SemaphoreType.DMA] * RB,
              [pltpu.SemaphoreType.DMA] * 2,
              [pltpu.SemaphoreType.DMA] * 3,
              [pltpu.SemaphoreType.DMA] * 2,
          ),
      ],
      compiler_params=pltpu.CompilerParams(
          use_tc_tiling_on_sc=False, needs_layout_passes=False),
  )
  return f(hq, s_dst, s_src)


def kernel(h_src, h_dst, edge_index):
  del h_dst
  src = edge_index[0].astype(jnp.int32)
  dst = edge_index[1].astype(jnp.int32)
  pad_i = jnp.arange(PAD, dtype=jnp.int32)
  pads = N + pad_i % 16
  s_dst = jnp.concatenate([dst, pads]).reshape(IROWS, CHUNK)
  s_src = jnp.concatenate([src, pads]).reshape(IROWS, CHUNK)
  hq = h_src.reshape(NQ * N, FQ)
  out_q = _run(hq, s_dst, s_src)
  return out_q.reshape(NQ, NP, FQ)[:, :N].transpose(1, 0, 2).reshape(N, D)


# speedup vs baseline: 1.1213x; 1.1213x over previous
import functools

import jax
import jax.numpy as jnp
from jax import lax
from jax.experimental import pallas as pl
from jax.experimental.pallas import tpu as pltpu
from jax.experimental.pallas import tpu_sc as plsc

N = 50000
D = 64
E = 800000

NQ = 4
FQ = 16
NT = 16
NC = 2

CHUNK = 128
CPB = 16
BLOCKS = 25
RB = 4
EPT = CHUNK * CPB * BLOCKS
EP = EPT * NT
PAD = EP - E
IROWS = EP // CHUNK

NP = 50048
NPT = NP // NT
WCHUNK = 136
NW = NPT // WCHUNK
GOFFS = tuple(range(0, WCHUNK - 15, 16)) + (WCHUNK - 16,)


def _sc_body(hq, is_dst, is_src, out_hbm,
             acc, rst, norm_d, norm_s, ib_g, ib_s, rows, ones, zb, inb,
             outb, nbuf, sems):
  cid = lax.axis_index("c")
  sid = lax.axis_index("s")
  sem_g, sem_f, sem_o, sem_i, sem_w, sem_z = sems

  def _init_ones(i, _):
    ones[i, :] = jnp.ones((FQ,), jnp.float32)
    return 0
  lax.fori_loop(0, CHUNK, _init_ones, 0)

  def _init_z(i, _):
    zb[i, :] = jnp.zeros((FQ,), jnp.float32)
    return 0
  lax.fori_loop(0, WCHUNK, _init_z, 0)

  for z in range(NW):
    pltpu.sync_copy(zb, acc.at[pl.ds(sid * NPT + z * WCHUNK, WCHUNK)])
    pltpu.sync_copy(zb, rst.at[pl.ds(sid * NPT + z * WCHUNK, WCHUNK)])
  plsc.subcore_barrier()

  zero16 = jnp.zeros((16,), jnp.int32)
  iota16 = lax.iota(jnp.int32, 16)

  def deg_prologue():
    def block(b, _):
      srow0 = sid * (BLOCKS * CPB) + b * CPB
      ci0 = pltpu.async_copy(is_dst.at[pl.ds(srow0, CPB)], ib_s, sem_i[0])
      ci1 = pltpu.async_copy(is_src.at[pl.ds(srow0, CPB)], ib_g, sem_i[1])
      ci0.wait()
      ci1.wait()
      scd = [None] * RB
      scs = [None] * RB
      for j in range(CPB):
        if scd[j % RB] is not None:
          scd[j % RB].wait()
        scd[j % RB] = pltpu.async_copy(
            ones, acc.at[ib_s.at[j]], sem_o[j % RB], add=True)
        if scs[j % RB] is not None:
          scs[j % RB].wait()
        scs[j % RB] = pltpu.async_copy(
            ones, rst.at[ib_g.at[j]], sem_f[j % RB], add=True)
      for x in scd + scs:
        if x is not None:
          x.wait()
      return 0
    lax.fori_loop(0, BLOCKS, block, 0)
    plsc.subcore_barrier()

    def compact(src_ref, norm_x, rezero):
      def body(w, _):
        node0 = sid * NPT + w * WCHUNK
        pltpu.sync_copy(src_ref.at[pl.ds(node0, WCHUNK)], inb.at[0])
        for g in GOFFS:
          dv = plsc.load_gather(inb, [zero16, iota16 + g, zero16])
          nbuf[pl.ds(g, 16)] = 1.0 / jnp.maximum(dv, 1.0)
        pltpu.sync_copy(nbuf, norm_x.at[pl.ds(node0, WCHUNK)])
        if rezero:
          pltpu.sync_copy(zb, src_ref.at[pl.ds(node0, WCHUNK)])
        return 0
      lax.fori_loop(0, NW, body, 0)

    compact(acc, norm_d, True)
    compact(rst, norm_s, False)
    plsc.subcore_barrier()

  deg_prologue()

  def stage(g_tbl, ig, is2, qb, addq):
    def xf(sl):
      if addq:
        ib_g[sl] = ib_g[sl] + qb

    def block(b, _):
      row0 = sid * (BLOCKS * CPB) + b * CPB
      ci0 = pltpu.async_copy(ig.at[pl.ds(row0, CPB)], ib_g, sem_i[0])
      ci1 = pltpu.async_copy(is2.at[pl.ds(row0, CPB)], ib_s, sem_i[1])
      ci0.wait()
      gcp = [None] * RB
      scf = [None] * RB
      for r in range(RB - 1):
        for k in range(CHUNK // 16):
          xf((r, pl.ds(k * 16, 16)))
        gcp[r] = pltpu.async_copy(g_tbl.at[ib_g.at[r]], rows.at[r], sem_g[r])
      for r in range(RB - 1, CPB):
        for k in range(CHUNK // 16):
          xf((r, pl.ds(k * 16, 16)))
      ci1.wait()
      for j in range(CPB):
        jn = j + RB - 1
        if jn < CPB:
          bn = jn % RB
          if scf[bn] is not None:
            scf[bn].wait()
          gcp[bn] = pltpu.async_copy(
              g_tbl.at[ib_g.at[jn]], rows.at[bn], sem_g[bn])
        gcp[j % RB].wait()
        scf[j % RB] = pltpu.async_copy(
            rows.at[j % RB], acc.at[ib_s.at[j]], sem_f[j % RB], add=True)
      for x in scf:
        if x is not None:
          x.wait()
      return 0
    lax.fori_loop(0, BLOCKS, block, 0)

  def writeout(dst_ref, norm_x, qb):
    node00 = sid * NPT
    icp = [None, None]
    ocp = None
    zcp = [None, None]
    icp[0] = pltpu.async_copy(
        acc.at[pl.ds(node00, WCHUNK)], inb.at[0], sem_w[0])
    for w in range(NW):
      pb = w % 2
      nb = (w + 1) % 2
      node0 = node00 + w * WCHUNK
      if w + 1 < NW:
        icp[nb] = pltpu.async_copy(
            acc.at[pl.ds(node0 + WCHUNK, WCHUNK)], inb.at[nb], sem_w[nb])
      pltpu.sync_copy(norm_x.at[pl.ds(node0, WCHUNK)], nbuf)
      icp[pb].wait()
      if ocp is not None:
        ocp.wait()
      def xform(i4, _):
        for u in range(4):
          i = 4 * i4 + u
          nv = plsc.load_gather(nbuf, [zero16 + i])
          outb[i, :] = inb[pb, i, :] * nv
        return 0
      lax.fori_loop(0, WCHUNK // 4, xform, 0)
      ocp = pltpu.async_copy(
          outb, dst_ref.at[pl.ds(qb + node0, WCHUNK)], sem_w[2])
      if zcp[pb] is not None:
        zcp[pb].wait()
      zcp[pb] = pltpu.async_copy(
          zb, acc.at[pl.ds(node0, WCHUNK)], sem_z[pb])
    ocp.wait()
    for x in zcp:
      if x is not None:
        x.wait()

  def one_pass(p, _):
    q = 2 * p + cid
    qb = q * NP
    stage(hq, is_src, is_dst, qb, True)
    plsc.subcore_barrier()
    writeout(rst, norm_d, 0)
    plsc.subcore_barrier()
    stage(rst, is_dst, is_src, 0, False)
    plsc.subcore_barrier()
    writeout(out_hbm, norm_s, qb)
    plsc.subcore_barrier()
    return 0

  lax.fori_loop(0, NC, one_pass, 0)


@functools.partial(jax.jit, static_argnames=())
def _run(hq, s_dst, s_src):
  mesh = plsc.VectorSubcoreMesh(core_axis_name="c", subcore_axis_name="s")
  f = pl.kernel(
      _sc_body,
      out_type=jax.ShapeDtypeStruct((NQ * NP, FQ), jnp.float32),
      mesh=mesh,
      scratch_types=[
          pltpu.VMEM_SHARED((NP, FQ), jnp.float32),
          pltpu.VMEM_SHARED((NP, FQ), jnp.float32),
          pltpu.VMEM_SHARED((NP,), jnp.float32),
          pltpu.VMEM_SHARED((NP,), jnp.float32),
          pltpu.VMEM((CPB, CHUNK), jnp.int32),
          pltpu.VMEM((CPB, CHUNK), jnp.int32),
          pltpu.VMEM((RB, CHUNK, FQ), jnp.float32),
          pltpu.VMEM((CHUNK, FQ), jnp.float32),
          pltpu.VMEM((WCHUNK, FQ), jnp.float32),
          pltpu.VMEM((2, WCHUNK, FQ), jnp.float32),
          pltpu.VMEM((WCHUNK, FQ), jnp.float32),
          pltpu.VMEM((WCHUNK,), jnp.float32),
          (
              [pltpu.SemaphoreType.DMA] * RB,
              [pltpu.SemaphoreType.DMA] * RB,
              [pltpu.SemaphoreType.DMA] * RB,
              [pltpu.SemaphoreType.DMA] * 2,
              [pltpu.SemaphoreType.DMA] * 3,
              [pltpu.SemaphoreType.DMA] * 2,
          ),
      ],
      compiler_params=pltpu.CompilerParams(
          use_tc_tiling_on_sc=False, needs_layout_passes=False),
  )
  return f(hq, s_dst, s_src)


def kernel(h_src, h_dst, edge_index):
  del h_dst
  src = edge_index[0].astype(jnp.int32)
  dst = edge_index[1].astype(jnp.int32)
  pad_i = jnp.arange(PAD, dtype=jnp.int32)
  pads = N + pad_i % 16
  s_dst = jnp.concatenate([dst, pads]).reshape(IROWS, CHUNK)
  s_src = jnp.concatenate([src, pads]).reshape(IROWS, CHUNK)
  hq = h_src.reshape(N, NQ, FQ).transpose(1, 0, 2)
  hq = jnp.pad(hq, ((0, 0), (0, NP - N), (0, 0))).reshape(NQ * NP, FQ)
  out_q = _run(hq, s_dst, s_src)
  return out_q.reshape(NQ, NP, FQ)[:, :N].transpose(1, 0, 2).reshape(N, D)
